# baseline (speedup 1.0000x reference)
import jax
import jax.numpy as jnp
from jax import lax
from jax.experimental import pallas as pl
from jax.experimental.pallas import tpu as pltpu
from jax.experimental.pallas import tpu_sc as plsc

N = 10000
E = 320000
G = 64
HID = 128
COND = 128
EF = 16
DEPTH = 6

NC = 2
NS = 16
NW = NC * NS
CH = 128
N_PAD = 10240
RPT = N_PAD // NS
RB = 1000
GRID_N = N // RB

K_BUF = 4
NHALF = 2
CHW = 80
HCHW = CHW // NHALF
E_PAD = CHW * CH * NW

_mesh = plsc.VectorSubcoreMesh(core_axis_name="c", subcore_axis_name="s")


def _make_gather_add():
    def body(tab_hbm, idx_hbm, scat_hbm, out_hbm, idx_v, scat_v, buf, acc):
        c = lax.axis_index("c")
        s = lax.axis_index("s")
        wid = s * NC + c
        base = s * RPT
        buf[...] = jnp.zeros((CH, HID), jnp.float32)
        for k in range(RPT // CH):
            pltpu.sync_copy(buf, acc.at[pl.ds(base + k * CH, CH)])
        plsc.subcore_barrier()

        for p in range(NHALF):
            pltpu.sync_copy(idx_hbm.at[wid, pl.ds(p * HCHW, HCHW)], idx_v)
            pltpu.sync_copy(scat_hbm.at[wid, pl.ds(p * HCHW, HCHW)], scat_v)

            def step(jl, carry):
                pltpu.sync_copy(tab_hbm.at[idx_v.at[jl]], buf)
                pltpu.sync_copy(buf, acc.at[scat_v.at[jl]], add=True)
                return carry

            lax.fori_loop(0, HCHW, step, 0)

        plsc.subcore_barrier()
        off = c * N_PAD + base
        for k in range(RPT // CH):
            pltpu.sync_copy(acc.at[pl.ds(base + k * CH, CH)], buf)
            pltpu.sync_copy(buf, out_hbm.at[pl.ds(off + k * CH, CH)])

    return pl.kernel(
        body,
        out_type=jax.ShapeDtypeStruct((NC * N_PAD, HID), jnp.float32),
        mesh=_mesh,
        scratch_types=[
            pltpu.VMEM((HCHW, CH), jnp.int32),
            pltpu.VMEM((HCHW, CH), jnp.int32),
            pltpu.VMEM((CH, HID), jnp.float32),
            pltpu.VMEM_SHARED((N_PAD, HID), jnp.float32),
        ],
    )


_gather_conv = _make_gather_add()


_B128 = pl.BlockSpec((RB, HID), lambda i: (i, 0))
_B16 = pl.BlockSpec((RB, EF), lambda i: (i, 0))
_BW = pl.BlockSpec((HID, HID), lambda i: (0, 0))
_BROW = pl.BlockSpec((1, HID), lambda i: (0, 0))
_BG = pl.BlockSpec((G, HID), lambda i: (0, 0))
_BNI = pl.BlockSpec((1, 1, RB), lambda i: (i, 0, 0))


def _tmlp_body(t_ref, wt1, bt1, wt2, bt2, wf, bf, gb_ref):
    i = lax.broadcasted_iota(jnp.int32, (G, 64), 1).astype(jnp.float32)
    freqs = jnp.exp(-jnp.log(1000.0) * i / 64.0)
    a = t_ref[...] * 1000.0 * freqs
    emb = jnp.concatenate([jnp.sin(a), jnp.cos(a)], axis=1)
    h = jnp.dot(emb, wt1[...], preferred_element_type=jnp.float32) + bt1[...]
    h = h * jax.nn.sigmoid(h)
    cond = jnp.dot(h, wt2[...], preferred_element_type=jnp.float32) + bt2[...]
    gb_ref[...] = jnp.dot(cond, wf[...], preferred_element_type=jnp.float32) + bf[...]


_tmlp = pl.pallas_call(
    _tmlp_body,
    out_shape=jax.ShapeDtypeStruct((G, DEPTH * 2 * HID), jnp.float32),
)


def _prep_body(a_ref, cc_ref, x_ref, ps0, ps1, q0, q1, win, bin_, we, be,
               y_ref, ex_ref, st_ref):
    ps = ps0[...] + ps1[...]
    deg_i = ps[:, EF:EF + 1]
    deg_o = (q0[...] + q1[...])[:, EF:EF + 1]
    ns = jnp.where(deg_o > 0, lax.rsqrt(jnp.maximum(deg_o, 1.0)), 0.0)
    nd = jnp.where(deg_i > 0, lax.rsqrt(jnp.maximum(deg_i, 1.0)), 0.0)
    inv_di = 1.0 / jnp.maximum(deg_i, 1.0)
    h0 = jnp.concatenate([a_ref[...], cc_ref[...], x_ref[...]], axis=1)
    y_ref[...] = jnp.dot(h0 * ns, win[...], preferred_element_type=jnp.float32)
    esum = jnp.dot(ps[:, :EF], we[...], preferred_element_type=jnp.float32)
    ex_ref[...] = (esum + deg_i * be[...]) * inv_di + bin_[...]
    st_ref[...] = jnp.concatenate(
        [ns, nd, jnp.zeros((RB, EF - 2), jnp.float32)], axis=1)


_prep = pl.pallas_call(
    _prep_body,
    grid=(GRID_N,),
    in_specs=[
        pl.BlockSpec((RB, 64), lambda i: (i, 0)),
        pl.BlockSpec((RB, 32), lambda i: (i, 0)),
        pl.BlockSpec((RB, 32), lambda i: (i, 0)),
        _B128, _B128, _B128, _B128,
        _BW,
        _BROW,
        pl.BlockSpec((EF, HID), lambda i: (0, 0)),
        _BROW,
    ],
    out_specs=[_B128, _B128, _B16],
    out_shape=[jax.ShapeDtypeStruct((N, HID), jnp.float32),
               jax.ShapeDtypeStruct((N, HID), jnp.float32),
               jax.ShapeDtypeStruct((N, EF), jnp.float32)],
)


def _combine0_body(p0, p1, ex, st, w, h_ref, y_ref):
    ns = st[...][:, 0:1]
    nd = st[...][:, 1:2]
    h = (p0[...] + p1[...]) * nd + ex[...]
    h_ref[...] = h
    y_ref[...] = jnp.dot(h * ns, w[...], preferred_element_type=jnp.float32)


_combine0 = pl.pallas_call(
    _combine0_body,
    grid=(GRID_N,),
    in_specs=[_B128, _B128, _B128, _B16, _BW],
    out_specs=[_B128, _B128],
    out_shape=[jax.ShapeDtypeStruct((N, HID), jnp.float32)] * 2,
)


def _make_combine_block(last):
    def body(p0, p1, hprev, bc, lw, lb, gamma, beta, nidx, st, *rest):
        if last:
            h_ref, pool_ref, cnt_ref = rest
        else:
            w = rest[0]
            h_ref, y_ref = rest[1:]
        ns = st[...][:, 0:1]
        nd = st[...][:, 1:2]
        hb = (p0[...] + p1[...]) * nd + bc[...]
        mu = jnp.mean(hb, axis=1, keepdims=True)
        var = jnp.mean((hb - mu) ** 2, axis=1, keepdims=True)
        hb = (hb - mu) * lax.rsqrt(var + 1e-5) * lw[...] + lb[...]
        oh_t = (lax.broadcasted_iota(jnp.int32, (G, RB), 0)
                == nidx[0]).astype(jnp.float32)
        dn = (((0,), (0,)), ((), ()))
        gn = lax.dot_general(oh_t, gamma[...], dn,
                             preferred_element_type=jnp.float32)
        bn = lax.dot_general(oh_t, beta[...], dn,
                             preferred_element_type=jnp.float32)
        hb = hb * (1.0 + gn) + bn
        hb = hb * jax.nn.sigmoid(hb)
        h = hb + hprev[...]
        h_ref[...] = h
        if last:
            pid = pl.program_id(0)

            @pl.when(pid == 0)
            def _():
                pool_ref[...] = jnp.zeros_like(pool_ref)
                cnt_ref[...] = jnp.zeros_like(cnt_ref)

            pool_ref[...] += jnp.dot(oh_t, h, preferred_element_type=jnp.float32)
            cnt_ref[...] += jnp.dot(oh_t, jnp.ones_like(h),
                                    preferred_element_type=jnp.float32)
        else:
            y_ref[...] = jnp.dot(h * ns, w[...], preferred_element_type=jnp.float32)

    base_in = [_B128, _B128, _B128, _BROW, _BROW, _BROW, _BG, _BG, _BNI, _B16]
    if last:
        return pl.pallas_call(
            body,
            grid=(GRID_N,),
            in_specs=base_in,
            out_specs=[_B128,
                       pl.BlockSpec((G, HID), lambda i: (0, 0)),
                       pl.BlockSpec((G, HID), lambda i: (0, 0))],
            out_shape=[jax.ShapeDtypeStruct((N, HID), jnp.float32),
                       jax.ShapeDtypeStruct((G, HID), jnp.float32),
                       jax.ShapeDtypeStruct((G, HID), jnp.float32)],
        )
    return pl.pallas_call(
        body,
        grid=(GRID_N,),
        in_specs=base_in + [_BW],
        out_specs=[_B128, _B128],
        out_shape=[jax.ShapeDtypeStruct((N, HID), jnp.float32)] * 2,
    )


_combine_mid = _make_combine_block(last=False)
_combine_last = _make_combine_block(last=True)


def _head_body(pool, cnt, wh, bh, o_ref):
    pooled = pool[...] / jnp.maximum(cnt[...], 1.0)
    o_ref[...] = jnp.sum(pooled * wh[...], axis=1, keepdims=True) + bh[...]


_head = pl.pallas_call(
    _head_body,
    out_shape=jax.ShapeDtypeStruct((G, 1), jnp.float32),
)


@jax.jit
def kernel(a_t, c_t, x_t, e_t, t, edge_index, n_index, W_in, b_in, W_e, b_e,
           W_t1, b_t1, W_t2, b_t2, blk_Wc, blk_bc, blk_lw, blk_lb, blk_Wf,
           blk_bf, W_head, b_head):
    src = edge_index[0]
    dst = edge_index[1]
    nidx3 = n_index.reshape(GRID_N, 1, RB)

    pad_a = jnp.zeros((E_PAD - E,), jnp.int32)
    pad_g = jnp.full((E_PAD - E,), N_PAD - 1, jnp.int32)
    src_g = jnp.concatenate([src, pad_a]).reshape(NW, CHW, CH)
    dst_g = jnp.concatenate([dst, pad_g]).reshape(NW, CHW, CH)
    src_scat = jnp.concatenate([src, pad_g]).reshape(NW, CHW, CH)

    epk = jnp.concatenate(
        [e_t, jnp.ones((E, 1), jnp.float32),
         jnp.zeros((E, HID - EF - 1), jnp.float32)], axis=1)
    epk = jnp.concatenate(
        [epk, jnp.zeros((E_PAD - E, HID), jnp.float32)], axis=0)
    ident = jnp.arange(E_PAD, dtype=jnp.int32).reshape(NW, CHW, CH)
    ps = _gather_conv(epk, ident, dst_g)
    qs = _gather_conv(epk, ident, src_scat)
    ps0, ps1 = ps[:N], ps[N_PAD:N_PAD + N]
    qs0, qs1 = qs[:N], qs[N_PAD:N_PAD + N]

    gb = _tmlp(t.reshape(G, 1), W_t1, b_t1.reshape(1, COND), W_t2,
               b_t2.reshape(1, COND),
               blk_Wf.transpose(1, 0, 2).reshape(COND, DEPTH * 2 * HID),
               blk_bf.reshape(1, DEPTH * 2 * HID))
    gb = gb.reshape(G, DEPTH, 2, HID).transpose(1, 2, 0, 3)
    gammas, betas = gb[:, 0], gb[:, 1]

    y, extra0, st = _prep(a_t, c_t, x_t, ps0, ps1, qs0, qs1,
                          W_in, b_in.reshape(1, HID), W_e, b_e.reshape(1, HID))

    p = _gather_conv(y, src_g, dst_g)
    h, y = _combine0(p[:N], p[N_PAD:N_PAD + N], extra0, st, blk_Wc[0])

    for i in range(DEPTH):
        p = _gather_conv(y, src_g, dst_g)
        args = (p[:N], p[N_PAD:N_PAD + N], h, blk_bc[i].reshape(1, HID),
                blk_lw[i].reshape(1, HID), blk_lb[i].reshape(1, HID),
                gammas[i], betas[i], nidx3, st)
        if i < DEPTH - 1:
            h, y = _combine_mid(*args, blk_Wc[i + 1])
        else:
            h, pool, cnt = _combine_last(*args)

    return _head(pool, cnt, W_head.reshape(1, HID), b_head.reshape(1, 1))


# speedup vs baseline: 1.2371x; 1.2371x over previous
import jax
import jax.numpy as jnp
from jax import lax
from jax.experimental import pallas as pl
from jax.experimental.pallas import tpu as pltpu
from jax.experimental.pallas import tpu_sc as plsc

N = 10000
E = 320000
G = 64
HID = 128
COND = 128
EF = 16
DEPTH = 6

NC = 2
NS = 16
NW = NC * NS
CH = 128
N_PAD = 10240
RPT = N_PAD // NS
RB = 1000
GRID_N = N // RB

CHW = 80
E_PAD = CHW * CH * NW

_mesh = plsc.VectorSubcoreMesh(core_axis_name="c", subcore_axis_name="s")


def _make_gather_add():
    G2 = CHW // 2

    def body(tab_hbm, idx_hbm, scat_hbm, out_hbm,
             i0, i1, s0, s1, r0, r1, m0, m1, acc):
        idxs = (i0, i1)
        scats = (s0, s1)
        rows = (r0, r1)
        sems = (m0, m1)
        c = lax.axis_index("c")
        s = lax.axis_index("s")
        wid = s * NC + c
        base = s * RPT
        r0[...] = jnp.zeros((CH, HID), jnp.float32)
        for k in range(RPT // CH):
            pltpu.sync_copy(r0, acc.at[pl.ds(base + k * CH, CH)])
        plsc.subcore_barrier()

        def load_idx(j, b):
            eb = (wid + j * NW) * CH
            pltpu.sync_copy(idx_hbm.at[pl.ds(eb, CH)], idxs[b])
            pltpu.sync_copy(scat_hbm.at[pl.ds(eb, CH)], scats[b])

        def start_g(b):
            pltpu.async_copy(tab_hbm.at[idxs[b]], rows[b], sems[b])

        def finish(b):
            pltpu.make_async_copy(tab_hbm.at[idxs[b]], rows[b], sems[b]).wait()
            pltpu.sync_copy(rows[b], acc.at[scats[b]], add=True)

        load_idx(0, 0)
        start_g(0)

        def step(g, carry):
            load_idx(2 * g + 1, 1)
            start_g(1)
            finish(0)
            load_idx(2 * g + 2, 0)
            start_g(0)
            finish(1)
            return carry

        lax.fori_loop(0, G2 - 1, step, 0)
        load_idx(CHW - 1, 1)
        start_g(1)
        finish(0)
        finish(1)

        plsc.subcore_barrier()
        off = c * N_PAD + base
        for k in range(RPT // CH):
            pltpu.sync_copy(acc.at[pl.ds(base + k * CH, CH)], r0)
            pltpu.sync_copy(r0, out_hbm.at[pl.ds(off + k * CH, CH)])

    return pl.kernel(
        body,
        out_type=jax.ShapeDtypeStruct((NC * N_PAD, HID), jnp.float32),
        mesh=_mesh,
        scratch_types=[
            pltpu.VMEM((CH,), jnp.int32),
            pltpu.VMEM((CH,), jnp.int32),
            pltpu.VMEM((CH,), jnp.int32),
            pltpu.VMEM((CH,), jnp.int32),
            pltpu.VMEM((CH, HID), jnp.float32),
            pltpu.VMEM((CH, HID), jnp.float32),
            pltpu.SemaphoreType.DMA,
            pltpu.SemaphoreType.DMA,
            pltpu.VMEM_SHARED((N_PAD, HID), jnp.float32),
        ],
    )


_gather_conv = _make_gather_add()


_B128 = pl.BlockSpec((RB, HID), lambda i: (i, 0))
_B16 = pl.BlockSpec((RB, EF), lambda i: (i, 0))
_BW = pl.BlockSpec((HID, HID), lambda i: (0, 0))
_BROW = pl.BlockSpec((1, HID), lambda i: (0, 0))
_BG = pl.BlockSpec((G, HID), lambda i: (0, 0))
_BNI = pl.BlockSpec((1, 1, RB), lambda i: (i, 0, 0))


def _tmlp_body(t_ref, wt1, bt1, wt2, bt2, wf, bf, gb_ref):
    i = lax.broadcasted_iota(jnp.int32, (G, 64), 1).astype(jnp.float32)
    freqs = jnp.exp(-jnp.log(1000.0) * i / 64.0)
    a = t_ref[...] * 1000.0 * freqs
    emb = jnp.concatenate([jnp.sin(a), jnp.cos(a)], axis=1)
    h = jnp.dot(emb, wt1[...], preferred_element_type=jnp.float32) + bt1[...]
    h = h * jax.nn.sigmoid(h)
    cond = jnp.dot(h, wt2[...], preferred_element_type=jnp.float32) + bt2[...]
    gb_ref[...] = jnp.dot(cond, wf[...], preferred_element_type=jnp.float32) + bf[...]


_tmlp = pl.pallas_call(
    _tmlp_body,
    out_shape=jax.ShapeDtypeStruct((G, DEPTH * 2 * HID), jnp.float32),
)


def _prep_body(a_ref, cc_ref, x_ref, ps0, ps1, q0, q1, win, bin_, we, be,
               y_ref, ex_ref, st_ref):
    ps = ps0[...] + ps1[...]
    deg_i = ps[:, EF:EF + 1]
    deg_o = (q0[...] + q1[...])[:, EF:EF + 1]
    ns = jnp.where(deg_o > 0, lax.rsqrt(jnp.maximum(deg_o, 1.0)), 0.0)
    nd = jnp.where(deg_i > 0, lax.rsqrt(jnp.maximum(deg_i, 1.0)), 0.0)
    inv_di = 1.0 / jnp.maximum(deg_i, 1.0)
    h0 = jnp.concatenate([a_ref[...], cc_ref[...], x_ref[...]], axis=1)
    y_ref[...] = jnp.dot(h0 * ns, win[...], preferred_element_type=jnp.float32)
    esum = jnp.dot(ps[:, :EF], we[...], preferred_element_type=jnp.float32)
    ex_ref[...] = (esum + deg_i * be[...]) * inv_di + bin_[...]
    st_ref[...] = jnp.concatenate(
        [ns, nd, jnp.zeros((RB, EF - 2), jnp.float32)], axis=1)


_prep = pl.pallas_call(
    _prep_body,
    grid=(GRID_N,),
    in_specs=[
        pl.BlockSpec((RB, 64), lambda i: (i, 0)),
        pl.BlockSpec((RB, 32), lambda i: (i, 0)),
        pl.BlockSpec((RB, 32), lambda i: (i, 0)),
        _B128, _B128, _B128, _B128,
        _BW,
        _BROW,
        pl.BlockSpec((EF, HID), lambda i: (0, 0)),
        _BROW,
    ],
    out_specs=[_B128, _B128, _B16],
    out_shape=[jax.ShapeDtypeStruct((N, HID), jnp.float32),
               jax.ShapeDtypeStruct((N, HID), jnp.float32),
               jax.ShapeDtypeStruct((N, EF), jnp.float32)],
)


def _combine0_body(p0, p1, ex, st, w, h_ref, y_ref):
    ns = st[...][:, 0:1]
    nd = st[...][:, 1:2]
    h = (p0[...] + p1[...]) * nd + ex[...]
    h_ref[...] = h
    y_ref[...] = jnp.dot(h * ns, w[...], preferred_element_type=jnp.float32)


_combine0 = pl.pallas_call(
    _combine0_body,
    grid=(GRID_N,),
    in_specs=[_B128, _B128, _B128, _B16, _BW],
    out_specs=[_B128, _B128],
    out_shape=[jax.ShapeDtypeStruct((N, HID), jnp.float32)] * 2,
)


def _make_combine_block(last):
    def body(p0, p1, hprev, bc, lw, lb, gamma, beta, nidx, st, *rest):
        if last:
            h_ref, pool_ref, cnt_ref = rest
        else:
            w = rest[0]
            h_ref, y_ref = rest[1:]
        ns = st[...][:, 0:1]
        nd = st[...][:, 1:2]
        hb = (p0[...] + p1[...]) * nd + bc[...]
        mu = jnp.mean(hb, axis=1, keepdims=True)
        var = jnp.mean((hb - mu) ** 2, axis=1, keepdims=True)
        hb = (hb - mu) * lax.rsqrt(var + 1e-5) * lw[...] + lb[...]
        oh_t = (lax.broadcasted_iota(jnp.int32, (G, RB), 0)
                == nidx[0]).astype(jnp.float32)
        dn = (((0,), (0,)), ((), ()))
        gn = lax.dot_general(oh_t, gamma[...], dn,
                             preferred_element_type=jnp.float32)
        bn = lax.dot_general(oh_t, beta[...], dn,
                             preferred_element_type=jnp.float32)
        hb = hb * (1.0 + gn) + bn
        hb = hb * jax.nn.sigmoid(hb)
        h = hb + hprev[...]
        h_ref[...] = h
        if last:
            pid = pl.program_id(0)

            @pl.when(pid == 0)
            def _():
                pool_ref[...] = jnp.zeros_like(pool_ref)
                cnt_ref[...] = jnp.zeros_like(cnt_ref)

            pool_ref[...] += jnp.dot(oh_t, h, preferred_element_type=jnp.float32)
            cnt_ref[...] += jnp.dot(oh_t, jnp.ones_like(h),
                                    preferred_element_type=jnp.float32)
        else:
            y_ref[...] = jnp.dot(h * ns, w[...], preferred_element_type=jnp.float32)

    base_in = [_B128, _B128, _B128, _BROW, _BROW, _BROW, _BG, _BG, _BNI, _B16]
    if last:
        return pl.pallas_call(
            body,
            grid=(GRID_N,),
            in_specs=base_in,
            out_specs=[_B128,
                       pl.BlockSpec((G, HID), lambda i: (0, 0)),
                       pl.BlockSpec((G, HID), lambda i: (0, 0))],
            out_shape=[jax.ShapeDtypeStruct((N, HID), jnp.float32),
                       jax.ShapeDtypeStruct((G, HID), jnp.float32),
                       jax.ShapeDtypeStruct((G, HID), jnp.float32)],
        )
    return pl.pallas_call(
        body,
        grid=(GRID_N,),
        in_specs=base_in + [_BW],
        out_specs=[_B128, _B128],
        out_shape=[jax.ShapeDtypeStruct((N, HID), jnp.float32)] * 2,
    )


_combine_mid = _make_combine_block(last=False)
_combine_last = _make_combine_block(last=True)


def _head_body(pool, cnt, wh, bh, o_ref):
    pooled = pool[...] / jnp.maximum(cnt[...], 1.0)
    o_ref[...] = jnp.sum(pooled * wh[...], axis=1, keepdims=True) + bh[...]


_head = pl.pallas_call(
    _head_body,
    out_shape=jax.ShapeDtypeStruct((G, 1), jnp.float32),
)


@jax.jit
def kernel(a_t, c_t, x_t, e_t, t, edge_index, n_index, W_in, b_in, W_e, b_e,
           W_t1, b_t1, W_t2, b_t2, blk_Wc, blk_bc, blk_lw, blk_lb, blk_Wf,
           blk_bf, W_head, b_head):
    src = edge_index[0]
    dst = edge_index[1]
    nidx3 = n_index.reshape(GRID_N, 1, RB)

    pad_a = jnp.zeros((E_PAD - E,), jnp.int32)
    pad_g = jnp.full((E_PAD - E,), N_PAD - 1, jnp.int32)
    src_g = jnp.concatenate([src, pad_a])
    dst_g = jnp.concatenate([dst, pad_g])

    epk = jnp.concatenate(
        [e_t, jnp.ones((E, 1), jnp.float32),
         jnp.zeros((E, HID - EF - 1), jnp.float32)], axis=1)
    epk = jnp.concatenate(
        [epk, jnp.zeros((E_PAD - E, HID), jnp.float32)], axis=0)
    ident = jnp.arange(E_PAD, dtype=jnp.int32)
    src_scat = jnp.concatenate([src, pad_g])
    ps = _gather_conv(epk, ident, dst_g)
    qs = _gather_conv(epk, ident, src_scat)
    ps0, ps1 = ps[:N], ps[N_PAD:N_PAD + N]
    qs0, qs1 = qs[:N], qs[N_PAD:N_PAD + N]

    gb = _tmlp(t.reshape(G, 1), W_t1, b_t1.reshape(1, COND), W_t2,
               b_t2.reshape(1, COND),
               blk_Wf.transpose(1, 0, 2).reshape(COND, DEPTH * 2 * HID),
               blk_bf.reshape(1, DEPTH * 2 * HID))
    gb = gb.reshape(G, DEPTH, 2, HID).transpose(1, 2, 0, 3)
    gammas, betas = gb[:, 0], gb[:, 1]

    y, extra0, st = _prep(a_t, c_t, x_t, ps0, ps1, qs0, qs1,
                          W_in, b_in.reshape(1, HID), W_e, b_e.reshape(1, HID))

    p = _gather_conv(y, src_g, dst_g)
    h, y = _combine0(p[:N], p[N_PAD:N_PAD + N], extra0, st, blk_Wc[0])

    for i in range(DEPTH):
        p = _gather_conv(y, src_g, dst_g)
        args = (p[:N], p[N_PAD:N_PAD + N], h, blk_bc[i].reshape(1, HID),
                blk_lw[i].reshape(1, HID), blk_lb[i].reshape(1, HID),
                gammas[i], betas[i], nidx3, st)
        if i < DEPTH - 1:
            h, y = _combine_mid(*args, blk_Wc[i + 1])
        else:
            h, pool, cnt = _combine_last(*args)

    return _head(pool, cnt, W_head.reshape(1, HID), b_head.reshape(1, 1))


# speedup vs baseline: 1.3418x; 1.0846x over previous
import jax
import jax.numpy as jnp
from jax import lax
from jax.experimental import pallas as pl
from jax.experimental.pallas import tpu as pltpu
from jax.experimental.pallas import tpu_sc as plsc

N = 10000
E = 320000
G = 64
HID = 128
COND = 128
EF = 16
DEPTH = 6

NC = 2
NS = 16
NW = NC * NS
CH = 128
N_PAD = 10240
RPT = N_PAD // NS
RB = 1000
GRID_N = N // RB

CHW_CONV = -(-E // (CH * NW))
E_PAD = CHW_CONV * CH * NW

_mesh = plsc.VectorSubcoreMesh(core_axis_name="c", subcore_axis_name="s")


def _make_gather_add(chw):
    def body(tab_hbm, idx_hbm, scat_hbm, out_hbm, idx_v, scat_v, rows_v, acc):
        c = lax.axis_index("c")
        s = lax.axis_index("s")
        wid = s * NC + c
        base = s * RPT
        rows_v[...] = jnp.zeros((CH, HID), jnp.float32)
        for k in range(RPT // CH):
            pltpu.sync_copy(rows_v, acc.at[pl.ds(base + k * CH, CH)])
        plsc.subcore_barrier()

        def step(i, carry):
            eb = (wid + i * NW) * CH
            pltpu.sync_copy(idx_hbm.at[pl.ds(eb, CH)], idx_v)
            pltpu.sync_copy(scat_hbm.at[pl.ds(eb, CH)], scat_v)
            pltpu.sync_copy(tab_hbm.at[idx_v], rows_v)
            pltpu.sync_copy(rows_v, acc.at[scat_v], add=True)
            return carry

        lax.fori_loop(0, chw, step, 0)
        plsc.subcore_barrier()
        off = c * N_PAD + base
        for k in range(RPT // CH):
            pltpu.sync_copy(acc.at[pl.ds(base + k * CH, CH)], rows_v)
            pltpu.sync_copy(rows_v, out_hbm.at[pl.ds(off + k * CH, CH)])

    return pl.kernel(
        body,
        out_type=jax.ShapeDtypeStruct((NC * N_PAD, HID), jnp.float32),
        mesh=_mesh,
        scratch_types=[
            pltpu.VMEM((CH,), jnp.int32),
            pltpu.VMEM((CH,), jnp.int32),
            pltpu.VMEM((CH, HID), jnp.float32),
            pltpu.VMEM_SHARED((N_PAD, HID), jnp.float32),
        ],
    )


_gather_conv = _make_gather_add(CHW_CONV)


_B128 = pl.BlockSpec((RB, HID), lambda i: (i, 0))
_B16 = pl.BlockSpec((RB, EF), lambda i: (i, 0))
_BW = pl.BlockSpec((HID, HID), lambda i: (0, 0))
_BROW = pl.BlockSpec((1, HID), lambda i: (0, 0))
_BG = pl.BlockSpec((G, HID), lambda i: (0, 0))
_BNI = pl.BlockSpec((1, 1, RB), lambda i: (i, 0, 0))


def _tmlp_body(t_ref, wt1, bt1, wt2, bt2, wf, bf, gb_ref):
    i = lax.broadcasted_iota(jnp.int32, (G, 64), 1).astype(jnp.float32)
    freqs = jnp.exp(-jnp.log(1000.0) * i / 64.0)
    a = t_ref[...] * 1000.0 * freqs
    emb = jnp.concatenate([jnp.sin(a), jnp.cos(a)], axis=1)
    h = jnp.dot(emb, wt1[...], preferred_element_type=jnp.float32) + bt1[...]
    h = h * jax.nn.sigmoid(h)
    cond = jnp.dot(h, wt2[...], preferred_element_type=jnp.float32) + bt2[...]
    gb_ref[...] = jnp.dot(cond, wf[...], preferred_element_type=jnp.float32) + bf[...]


_tmlp = pl.pallas_call(
    _tmlp_body,
    out_shape=jax.ShapeDtypeStruct((G, DEPTH * 2 * HID), jnp.float32),
)


def _prep_body(a_ref, cc_ref, x_ref, ps0, ps1, q0, q1, win, bin_, we, be,
               y_ref, ex_ref, st_ref):
    ps = ps0[...] + ps1[...]
    deg_i = ps[:, EF:EF + 1]
    deg_o = (q0[...] + q1[...])[:, EF:EF + 1]
    ns = jnp.where(deg_o > 0, lax.rsqrt(jnp.maximum(deg_o, 1.0)), 0.0)
    nd = jnp.where(deg_i > 0, lax.rsqrt(jnp.maximum(deg_i, 1.0)), 0.0)
    inv_di = 1.0 / jnp.maximum(deg_i, 1.0)
    h0 = jnp.concatenate([a_ref[...], cc_ref[...], x_ref[...]], axis=1)
    y_ref[...] = jnp.dot(h0 * ns, win[...], preferred_element_type=jnp.float32)
    esum = jnp.dot(ps[:, :EF], we[...], preferred_element_type=jnp.float32)
    ex_ref[...] = (esum + deg_i * be[...]) * inv_di + bin_[...]
    st_ref[...] = jnp.concatenate(
        [ns, nd, jnp.zeros((RB, EF - 2), jnp.float32)], axis=1)


_prep = pl.pallas_call(
    _prep_body,
    grid=(GRID_N,),
    in_specs=[
        pl.BlockSpec((RB, 64), lambda i: (i, 0)),
        pl.BlockSpec((RB, 32), lambda i: (i, 0)),
        pl.BlockSpec((RB, 32), lambda i: (i, 0)),
        _B128, _B128, _B128, _B128,
        _BW,
        _BROW,
        pl.BlockSpec((EF, HID), lambda i: (0, 0)),
        _BROW,
    ],
    out_specs=[_B128, _B128, _B16],
    out_shape=[jax.ShapeDtypeStruct((N, HID), jnp.float32),
               jax.ShapeDtypeStruct((N, HID), jnp.float32),
               jax.ShapeDtypeStruct((N, EF), jnp.float32)],
)


def _combine0_body(p0, p1, ex, st, w, h_ref, y_ref):
    ns = st[...][:, 0:1]
    nd = st[...][:, 1:2]
    h = (p0[...] + p1[...]) * nd + ex[...]
    h_ref[...] = h
    y_ref[...] = jnp.dot(h * ns, w[...], preferred_element_type=jnp.float32)


_combine0 = pl.pallas_call(
    _combine0_body,
    grid=(GRID_N,),
    in_specs=[_B128, _B128, _B128, _B16, _BW],
    out_specs=[_B128, _B128],
    out_shape=[jax.ShapeDtypeStruct((N, HID), jnp.float32)] * 2,
)


def _make_combine_block(last):
    def body(p0, p1, hprev, bc, lw, lb, gamma, beta, nidx, st, *rest):
        if last:
            h_ref, pool_ref, cnt_ref = rest
        else:
            w = rest[0]
            h_ref, y_ref = rest[1:]
        ns = st[...][:, 0:1]
        nd = st[...][:, 1:2]
        hb = (p0[...] + p1[...]) * nd + bc[...]
        mu = jnp.mean(hb, axis=1, keepdims=True)
        var = jnp.mean((hb - mu) ** 2, axis=1, keepdims=True)
        hb = (hb - mu) * lax.rsqrt(var + 1e-5) * lw[...] + lb[...]
        oh_t = (lax.broadcasted_iota(jnp.int32, (G, RB), 0)
                == nidx[0]).astype(jnp.float32)
        dn = (((0,), (0,)), ((), ()))
        gn = lax.dot_general(oh_t, gamma[...], dn,
                             preferred_element_type=jnp.float32)
        bn = lax.dot_general(oh_t, beta[...], dn,
                             preferred_element_type=jnp.float32)
        hb = hb * (1.0 + gn) + bn
        hb = hb * jax.nn.sigmoid(hb)
        h = hb + hprev[...]
        h_ref[...] = h
        if last:
            pid = pl.program_id(0)

            @pl.when(pid == 0)
            def _():
                pool_ref[...] = jnp.zeros_like(pool_ref)
                cnt_ref[...] = jnp.zeros_like(cnt_ref)

            pool_ref[...] += jnp.dot(oh_t, h, preferred_element_type=jnp.float32)
            cnt_ref[...] += jnp.dot(oh_t, jnp.ones_like(h),
                                    preferred_element_type=jnp.float32)
        else:
            y_ref[...] = jnp.dot(h * ns, w[...], preferred_element_type=jnp.float32)

    base_in = [_B128, _B128, _B128, _BROW, _BROW, _BROW, _BG, _BG, _BNI, _B16]
    if last:
        return pl.pallas_call(
            body,
            grid=(GRID_N,),
            in_specs=base_in,
            out_specs=[_B128,
                       pl.BlockSpec((G, HID), lambda i: (0, 0)),
                       pl.BlockSpec((G, HID), lambda i: (0, 0))],
            out_shape=[jax.ShapeDtypeStruct((N, HID), jnp.float32),
                       jax.ShapeDtypeStruct((G, HID), jnp.float32),
                       jax.ShapeDtypeStruct((G, HID), jnp.float32)],
        )
    return pl.pallas_call(
        body,
        grid=(GRID_N,),
        in_specs=base_in + [_BW],
        out_specs=[_B128, _B128],
        out_shape=[jax.ShapeDtypeStruct((N, HID), jnp.float32)] * 2,
    )


_combine_mid = _make_combine_block(last=False)
_combine_last = _make_combine_block(last=True)


def _head_body(pool, cnt, wh, bh, o_ref):
    pooled = pool[...] / jnp.maximum(cnt[...], 1.0)
    o_ref[...] = jnp.sum(pooled * wh[...], axis=1, keepdims=True) + bh[...]


_head = pl.pallas_call(
    _head_body,
    out_shape=jax.ShapeDtypeStruct((G, 1), jnp.float32),
)


@jax.jit
def kernel(a_t, c_t, x_t, e_t, t, edge_index, n_index, W_in, b_in, W_e, b_e,
           W_t1, b_t1, W_t2, b_t2, blk_Wc, blk_bc, blk_lw, blk_lb, blk_Wf,
           blk_bf, W_head, b_head):
    src = edge_index[0]
    dst = edge_index[1]
    nidx3 = n_index.reshape(GRID_N, 1, RB)

    pad_a = jnp.zeros((E_PAD - E,), jnp.int32)
    pad_g = jnp.full((E_PAD - E,), N_PAD - 1, jnp.int32)
    src_g = jnp.concatenate([src, pad_a])
    dst_g = jnp.concatenate([dst, pad_g])

    epk = jnp.concatenate(
        [e_t, jnp.ones((E, 1), jnp.float32),
         jnp.zeros((E, HID - EF - 1), jnp.float32)], axis=1)
    epk = jnp.concatenate(
        [epk, jnp.zeros((E_PAD - E, HID), jnp.float32)], axis=0)
    ident = jnp.arange(E_PAD, dtype=jnp.int32)
    src_scat = jnp.concatenate([src, pad_g])
    ps = _gather_conv(epk, ident, dst_g)
    qs = _gather_conv(epk, ident, src_scat)
    ps0, ps1 = ps[:N], ps[N_PAD:N_PAD + N]
    qs0, qs1 = qs[:N], qs[N_PAD:N_PAD + N]

    gb = _tmlp(t.reshape(G, 1), W_t1, b_t1.reshape(1, COND), W_t2,
               b_t2.reshape(1, COND),
               blk_Wf.transpose(1, 0, 2).reshape(COND, DEPTH * 2 * HID),
               blk_bf.reshape(1, DEPTH * 2 * HID))
    gb = gb.reshape(G, DEPTH, 2, HID).transpose(1, 2, 0, 3)
    gammas, betas = gb[:, 0], gb[:, 1]

    y, extra0, st = _prep(a_t, c_t, x_t, ps0, ps1, qs0, qs1,
                          W_in, b_in.reshape(1, HID), W_e, b_e.reshape(1, HID))

    p = _gather_conv(y, src_g, dst_g)
    h, y = _combine0(p[:N], p[N_PAD:N_PAD + N], extra0, st, blk_Wc[0])

    for i in range(DEPTH):
        p = _gather_conv(y, src_g, dst_g)
        args = (p[:N], p[N_PAD:N_PAD + N], h, blk_bc[i].reshape(1, HID),
                blk_lw[i].reshape(1, HID), blk_lb[i].reshape(1, HID),
                gammas[i], betas[i], nidx3, st)
        if i < DEPTH - 1:
            h, y = _combine_mid(*args, blk_Wc[i + 1])
        else:
            h, pool, cnt = _combine_last(*args)

    return _head(pool, cnt, W_head.reshape(1, HID), b_head.reshape(1, 1))
